# Initial kernel scaffold; baseline (speedup 1.0000x reference)
import jax
import jax.numpy as jnp
from jax.experimental import pallas as pl


def kernel(q, k, v, W1, b1, W2, b2, in_proj_w, in_proj_b, out_w, out_b):
    raise NotImplementedError("write your pallas kernel here")


# speedup vs baseline: 1.2069x; 1.2069x over previous
import math

import jax
import jax.numpy as jnp
from jax import lax
from jax.experimental import pallas as pl

H = 4
SPARSITY_RATIO = 0.1
MEMORY_PRESSURE = 0.5
THRESH = SPARSITY_RATIO * (1.0 + MEMORY_PRESSURE)
NEG = -1e30
BQ = 256


def _proj_kernel(k_ref, v_ref, q_ref, wkt_ref, bk_ref, wvt_ref, bv_ref,
                 w1t_ref, b1_ref, w2t_ref, b2_ref,
                 kp_ref, vp_ref, imp_ref):
    kp_ref[...] = k_ref[...] @ wkt_ref[...] + bk_ref[...]
    vp_ref[...] = v_ref[...] @ wvt_ref[...] + bv_ref[...]
    hid = jnp.maximum(q_ref[...] @ w1t_ref[...] + b1_ref[...], 0.0)
    logit = hid @ w2t_ref[...] + b2_ref[...]
    imp_ref[...] = jax.nn.sigmoid(logit)


def _attn_kernel(n_tokens, q_ref, impt_ref, imp_ref, kp_ref, vp_ref,
                 wqt_ref, bq_ref, owt_ref, ob_ref, out_ref):
    i = pl.program_id(0)
    hd = kp_ref.shape[1] // H

    imp_row = impt_ref[...]
    validk = (imp_row > THRESH).astype(jnp.float32)
    count = jnp.sum(validk)
    use_fb = count == 0.0
    fb_row = (lax.broadcasted_iota(jnp.int32, imp_row.shape, 1) < 32
              ).astype(jnp.float32)
    validk = jnp.where(use_fb, fb_row, validk)
    kbias = (validk - 1.0) * (-NEG)

    impq = imp_ref[...]
    rows = lax.broadcasted_iota(jnp.int32, impq.shape, 0) + i * BQ
    validq = jnp.where(use_fb, (rows < 32).astype(jnp.float32),
                       (impq > THRESH).astype(jnp.float32))

    qp = q_ref[...] @ wqt_ref[...] + bq_ref[...]
    scale = 1.0 / math.sqrt(hd)
    acc = jnp.zeros(out_ref.shape, jnp.float32)
    for h in range(H):
        sl = slice(h * hd, (h + 1) * hd)
        qh = qp[:, sl]
        s = lax.dot_general(qh, kp_ref[:, sl], (((1,), (1,)), ((), ())),
                            preferred_element_type=jnp.float32)
        s = s * scale + kbias
        m = jnp.max(s, axis=1, keepdims=True)
        p = jnp.exp(s - m)
        l = jnp.sum(p, axis=1, keepdims=True)
        ctx = lax.dot_general(p / l, vp_ref[:, sl],
                              (((1,), (0,)), ((), ())),
                              preferred_element_type=jnp.float32)
        acc = acc + lax.dot_general(ctx, owt_ref[sl, :],
                                    (((1,), (0,)), ((), ())),
                                    preferred_element_type=jnp.float32)
    out_ref[...] = (acc + ob_ref[...]) * validq


def kernel(q, k, v, W1, b1, W2, b2, in_proj_w, in_proj_b, out_w, out_b):
    batch, seq, d = q.shape
    n = batch * seq
    nblk = n // BQ
    dh = W1.shape[0]

    q2 = q.reshape(n, d)
    k2 = k.reshape(n, d)
    v2 = v.reshape(n, d)
    Wq, Wk, Wv = jnp.split(in_proj_w, 3, axis=0)
    bq, bk, bv = jnp.split(in_proj_b, 3)
    row = lambda x: x.reshape(1, -1)

    full = lambda shape: pl.BlockSpec(shape, lambda i: (0,) * len(shape))
    blk = pl.BlockSpec((BQ, d), lambda i: (i, 0))

    kp, vp, imp = pl.pallas_call(
        _proj_kernel,
        grid=(nblk,),
        in_specs=[blk, blk, blk,
                  full((d, d)), full((1, d)), full((d, d)), full((1, d)),
                  full((d, dh)), full((1, dh)), full((dh, 1)), full((1, 1))],
        out_specs=[blk, blk, pl.BlockSpec((BQ, 1), lambda i: (i, 0))],
        out_shape=[jax.ShapeDtypeStruct((n, d), jnp.float32),
                   jax.ShapeDtypeStruct((n, d), jnp.float32),
                   jax.ShapeDtypeStruct((n, 1), jnp.float32)],
    )(k2, v2, q2, Wk.T, row(bk), Wv.T, row(bv),
      W1.T, row(b1), W2.T, row(b2))

    impt = imp.reshape(1, n)

    out = pl.pallas_call(
        lambda *refs: _attn_kernel(n, *refs),
        grid=(nblk,),
        in_specs=[blk, full((1, n)), pl.BlockSpec((BQ, 1), lambda i: (i, 0)),
                  full((n, d)), full((n, d)),
                  full((d, d)), full((1, d)), full((d, d)), full((1, d))],
        out_specs=blk,
        out_shape=jax.ShapeDtypeStruct((n, d), jnp.float32),
    )(q2, impt, imp, kp, vp, Wq.T, row(bq), out_w.T, row(out_b))

    return out.reshape(batch, seq, d)


# speedup vs baseline: 1.2996x; 1.0768x over previous
import math

import jax
import jax.numpy as jnp
from jax import lax
from jax.experimental import pallas as pl

H = 4
SPARSITY_RATIO = 0.1
MEMORY_PRESSURE = 0.5
THRESH = SPARSITY_RATIO * (1.0 + MEMORY_PRESSURE)
NEG = -1e30
BQ = 512


def _proj_kernel(k_ref, v_ref, q_ref, wkt_ref, bk_ref, wvt_ref, bv_ref,
                 w1t_ref, b1_ref, w2t_ref, b2_ref,
                 kp_ref, vp_ref, imp_ref):
    kp_ref[...] = k_ref[...] @ wkt_ref[...] + bk_ref[...]
    vp_ref[...] = v_ref[...] @ wvt_ref[...] + bv_ref[...]
    hid = jnp.maximum(q_ref[...] @ w1t_ref[...] + b1_ref[...], 0.0)
    logit = hid @ w2t_ref[...] + b2_ref[...]
    imp_ref[...] = jax.nn.sigmoid(logit)


def _attn_kernel(n_tokens, q_ref, impt_ref, imp_ref, kp_ref, vp_ref,
                 wqt_ref, bq_ref, owt_ref, ob_ref, out_ref):
    i = pl.program_id(0)
    hd = kp_ref.shape[1] // H

    imp_row = impt_ref[...]
    validk = (imp_row > THRESH).astype(jnp.float32)
    count = jnp.sum(validk)
    use_fb = count == 0.0
    fb_row = (lax.broadcasted_iota(jnp.int32, imp_row.shape, 1) < 32
              ).astype(jnp.float32)
    validk = jnp.where(use_fb, fb_row, validk)
    kbias = (validk - 1.0) * (-NEG)

    impq = imp_ref[...]
    rows = lax.broadcasted_iota(jnp.int32, impq.shape, 0) + i * BQ
    validq = jnp.where(use_fb, (rows < 32).astype(jnp.float32),
                       (impq > THRESH).astype(jnp.float32))

    scale = 1.0 / math.sqrt(hd)
    qp = (q_ref[...] @ wqt_ref[...] + bq_ref[...]) * scale
    acc = jnp.zeros(out_ref.shape, jnp.float32)
    for h in range(H):
        sl = slice(h * hd, (h + 1) * hd)
        qh = qp[:, sl]
        s = lax.dot_general(qh, kp_ref[:, sl], (((1,), (1,)), ((), ())),
                            preferred_element_type=jnp.float32)
        s = s + kbias
        m = jnp.max(s, axis=1, keepdims=True)
        p = jnp.exp(s - m)
        l = jnp.sum(p, axis=1, keepdims=True)
        ctx = lax.dot_general(p, vp_ref[:, sl],
                              (((1,), (0,)), ((), ())),
                              preferred_element_type=jnp.float32) / l
        acc = acc + lax.dot_general(ctx, owt_ref[sl, :],
                                    (((1,), (0,)), ((), ())),
                                    preferred_element_type=jnp.float32)
    out_ref[...] = (acc + ob_ref[...]) * validq


def kernel(q, k, v, W1, b1, W2, b2, in_proj_w, in_proj_b, out_w, out_b):
    batch, seq, d = q.shape
    n = batch * seq
    nblk = n // BQ
    dh = W1.shape[0]

    q2 = q.reshape(n, d)
    k2 = k.reshape(n, d)
    v2 = v.reshape(n, d)
    Wq, Wk, Wv = jnp.split(in_proj_w, 3, axis=0)
    bq, bk, bv = jnp.split(in_proj_b, 3)
    row = lambda x: x.reshape(1, -1)

    full = lambda shape: pl.BlockSpec(shape, lambda i: (0,) * len(shape))
    blk = pl.BlockSpec((BQ, d), lambda i: (i, 0))

    kp, vp, imp = pl.pallas_call(
        _proj_kernel,
        grid=(nblk,),
        in_specs=[blk, blk, blk,
                  full((d, d)), full((1, d)), full((d, d)), full((1, d)),
                  full((d, dh)), full((1, dh)), full((dh, 1)), full((1, 1))],
        out_specs=[blk, blk, pl.BlockSpec((BQ, 1), lambda i: (i, 0))],
        out_shape=[jax.ShapeDtypeStruct((n, d), jnp.float32),
                   jax.ShapeDtypeStruct((n, d), jnp.float32),
                   jax.ShapeDtypeStruct((n, 1), jnp.float32)],
    )(k2, v2, q2, Wk.T, row(bk), Wv.T, row(bv),
      W1.T, row(b1), W2.T, row(b2))

    impt = imp.reshape(1, n)

    out = pl.pallas_call(
        lambda *refs: _attn_kernel(n, *refs),
        grid=(nblk,),
        in_specs=[blk, full((1, n)), pl.BlockSpec((BQ, 1), lambda i: (i, 0)),
                  full((n, d)), full((n, d)),
                  full((d, d)), full((1, d)), full((d, d)), full((1, d))],
        out_specs=blk,
        out_shape=jax.ShapeDtypeStruct((n, d), jnp.float32),
    )(q2, impt, imp, kp, vp, Wq.T, row(bq), out_w.T, row(out_b))

    return out.reshape(batch, seq, d)


# speedup vs baseline: 1.3051x; 1.0042x over previous
import math

import jax
import jax.numpy as jnp
from jax import lax
from jax.experimental import pallas as pl

H = 4
SPARSITY_RATIO = 0.1
MEMORY_PRESSURE = 0.5
THRESH = SPARSITY_RATIO * (1.0 + MEMORY_PRESSURE)
NEG = -1e30
BQ = 512


def _proj_kernel(k_ref, v_ref, q_ref, wkt_ref, bk_ref, wvt_ref, bv_ref,
                 w1t_ref, b1_ref, w2t_ref, b2_ref,
                 kp_ref, vp_ref, imp_ref):
    kp_ref[...] = k_ref[...] @ wkt_ref[...] + bk_ref[...]
    vp_ref[...] = v_ref[...] @ wvt_ref[...] + bv_ref[...]
    hid = jnp.maximum(q_ref[...] @ w1t_ref[...] + b1_ref[...], 0.0)
    logit = hid @ w2t_ref[...] + b2_ref[...]
    imp_ref[...] = jax.nn.sigmoid(logit)


def _attn_kernel(n_tokens, q_ref, impt_ref, imp_ref, kp_ref, vp_ref,
                 wqt_ref, bq_ref, owt_ref, ob_ref, out_ref):
    i = pl.program_id(0)
    hd = kp_ref.shape[1] // H

    imp_row = impt_ref[...]
    validk = (imp_row > THRESH).astype(jnp.float32)
    count = jnp.sum(validk)
    use_fb = count == 0.0
    fb_row = (lax.broadcasted_iota(jnp.int32, imp_row.shape, 1) < 32
              ).astype(jnp.float32)
    validk = jnp.where(use_fb, fb_row, validk)
    kbias = (validk - 1.0) * (-NEG)

    impq = imp_ref[...]
    rows = lax.broadcasted_iota(jnp.int32, impq.shape, 0) + i * BQ
    validq = jnp.where(use_fb, (rows < 32).astype(jnp.float32),
                       (impq > THRESH).astype(jnp.float32))

    scale = 1.0 / math.sqrt(hd)
    qp = (q_ref[...] @ wqt_ref[...] + bq_ref[...]) * scale
    acc = jnp.zeros(out_ref.shape, jnp.float32)
    for h in range(H):
        sl = slice(h * hd, (h + 1) * hd)
        qh = qp[:, sl].astype(jnp.bfloat16)
        kh = kp_ref[:, sl].astype(jnp.bfloat16)
        s = lax.dot_general(qh, kh, (((1,), (1,)), ((), ())),
                            preferred_element_type=jnp.float32)
        s = s + kbias
        m = jnp.max(s, axis=1, keepdims=True)
        p = jnp.exp(s - m)
        l = jnp.sum(p, axis=1, keepdims=True)
        ctx = lax.dot_general(p.astype(jnp.bfloat16),
                              vp_ref[:, sl].astype(jnp.bfloat16),
                              (((1,), (0,)), ((), ())),
                              preferred_element_type=jnp.float32) / l
        acc = acc + lax.dot_general(ctx, owt_ref[sl, :],
                                    (((1,), (0,)), ((), ())),
                                    preferred_element_type=jnp.float32)
    out_ref[...] = (acc + ob_ref[...]) * validq


def kernel(q, k, v, W1, b1, W2, b2, in_proj_w, in_proj_b, out_w, out_b):
    batch, seq, d = q.shape
    n = batch * seq
    nblk = n // BQ
    dh = W1.shape[0]

    q2 = q.reshape(n, d)
    k2 = k.reshape(n, d)
    v2 = v.reshape(n, d)
    Wq, Wk, Wv = jnp.split(in_proj_w, 3, axis=0)
    bq, bk, bv = jnp.split(in_proj_b, 3)
    row = lambda x: x.reshape(1, -1)

    full = lambda shape: pl.BlockSpec(shape, lambda i: (0,) * len(shape))
    blk = pl.BlockSpec((BQ, d), lambda i: (i, 0))

    kp, vp, imp = pl.pallas_call(
        _proj_kernel,
        grid=(nblk,),
        in_specs=[blk, blk, blk,
                  full((d, d)), full((1, d)), full((d, d)), full((1, d)),
                  full((d, dh)), full((1, dh)), full((dh, 1)), full((1, 1))],
        out_specs=[blk, blk, pl.BlockSpec((BQ, 1), lambda i: (i, 0))],
        out_shape=[jax.ShapeDtypeStruct((n, d), jnp.float32),
                   jax.ShapeDtypeStruct((n, d), jnp.float32),
                   jax.ShapeDtypeStruct((n, 1), jnp.float32)],
    )(k2, v2, q2, Wk.T, row(bk), Wv.T, row(bv),
      W1.T, row(b1), W2.T, row(b2))

    impt = imp.reshape(1, n)

    out = pl.pallas_call(
        lambda *refs: _attn_kernel(n, *refs),
        grid=(nblk,),
        in_specs=[blk, full((1, n)), pl.BlockSpec((BQ, 1), lambda i: (i, 0)),
                  full((n, d)), full((n, d)),
                  full((d, d)), full((1, d)), full((d, d)), full((1, d))],
        out_specs=blk,
        out_shape=jax.ShapeDtypeStruct((n, d), jnp.float32),
    )(q2, impt, imp, kp, vp, Wq.T, row(bq), out_w.T, row(out_b))

    return out.reshape(batch, seq, d)


# speedup vs baseline: 1.8032x; 1.3816x over previous
import math

import jax
import jax.numpy as jnp
from jax import lax
from jax.experimental import pallas as pl

H = 4
SPARSITY_RATIO = 0.1
MEMORY_PRESSURE = 0.5
THRESH = SPARSITY_RATIO * (1.0 + MEMORY_PRESSURE)
NEG = -1e30
BQ = 512

_DNT = (((1,), (1,)), ((), ()))


def _mmt(x, w):
    return lax.dot_general(x, w, _DNT, preferred_element_type=jnp.float32)


def _proj_kernel(k_ref, v_ref, q_ref, wk_ref, wv_ref, bk_ref, bv_ref,
                 w1_ref, b1_ref, w2_ref, b2_ref,
                 kp_ref, vp_ref, imp_ref):
    kp_ref[...] = _mmt(k_ref[...], wk_ref[...]) + bk_ref[0]
    vp_ref[...] = _mmt(v_ref[...], wv_ref[...]) + bv_ref[0]
    hid = jnp.maximum(_mmt(q_ref[...], w1_ref[...]) + b1_ref[...], 0.0)
    logit = jnp.sum(hid * w2_ref[...], axis=1, keepdims=True) + b2_ref[...]
    imp_ref[...] = jax.nn.sigmoid(logit)


def _attn_kernel(q_ref, impt_ref, imp_ref, kp_ref, vp_ref,
                 wq_ref, bq_ref, ow_ref, ob_ref, out_ref):
    i = pl.program_id(0)
    hd = kp_ref.shape[1] // H

    imp_row = impt_ref[...]
    validk = (imp_row > THRESH).astype(jnp.float32)
    count = jnp.sum(validk)
    use_fb = count == 0.0
    fb_row = (lax.broadcasted_iota(jnp.int32, imp_row.shape, 1) < 32
              ).astype(jnp.float32)
    validk = jnp.where(use_fb, fb_row, validk)
    kbias = (validk - 1.0) * (-NEG)

    impq = imp_ref[...]
    rows = lax.broadcasted_iota(jnp.int32, impq.shape, 0) + i * BQ
    validq = jnp.where(use_fb, (rows < 32).astype(jnp.float32),
                       (impq > THRESH).astype(jnp.float32))

    scale = 1.0 / math.sqrt(hd)
    qp = (_mmt(q_ref[...], wq_ref[...]) + bq_ref[0]) * scale
    acc = jnp.zeros(out_ref.shape, jnp.float32)
    for h in range(H):
        sl = slice(h * hd, (h + 1) * hd)
        qh = qp[:, sl].astype(jnp.bfloat16)
        kh = kp_ref[:, sl].astype(jnp.bfloat16)
        s = lax.dot_general(qh, kh, _DNT,
                            preferred_element_type=jnp.float32)
        s = s + kbias
        m = jnp.max(s, axis=1, keepdims=True)
        p = jnp.exp(s - m)
        l = jnp.sum(p, axis=1, keepdims=True)
        ctx = lax.dot_general(p.astype(jnp.bfloat16),
                              vp_ref[:, sl].astype(jnp.bfloat16),
                              (((1,), (0,)), ((), ())),
                              preferred_element_type=jnp.float32) / l
        acc = acc + _mmt(ctx, ow_ref[:, sl])
    out_ref[...] = (acc + ob_ref[...]) * validq


def kernel(q, k, v, W1, b1, W2, b2, in_proj_w, in_proj_b, out_w, out_b):
    batch, seq, d = q.shape
    n = batch * seq
    nblk = n // BQ
    dh = W1.shape[0]

    q2 = q.reshape(n, d)
    k2 = k.reshape(n, d)
    v2 = v.reshape(n, d)
    ipb = in_proj_b.reshape(3, 1, d)

    full = lambda shape: pl.BlockSpec(shape, lambda i: (0,) * len(shape))
    blk = pl.BlockSpec((BQ, d), lambda i: (i, 0))
    ipw_at = lambda j: pl.BlockSpec((d, d), lambda i: (j, 0))
    ipb_at = lambda j: pl.BlockSpec((1, 1, d), lambda i: (j, 0, 0))

    kp, vp, imp = pl.pallas_call(
        _proj_kernel,
        grid=(nblk,),
        in_specs=[blk, blk, blk,
                  ipw_at(1), ipw_at(2), ipb_at(1), ipb_at(2),
                  full((dh, d)), full((1, dh)), full((1, dh)), full((1, 1))],
        out_specs=[blk, blk, pl.BlockSpec((BQ, 1), lambda i: (i, 0))],
        out_shape=[jax.ShapeDtypeStruct((n, d), jnp.float32),
                   jax.ShapeDtypeStruct((n, d), jnp.float32),
                   jax.ShapeDtypeStruct((n, 1), jnp.float32)],
    )(k2, v2, q2, in_proj_w, in_proj_w, ipb, ipb,
      W1, b1.reshape(1, dh), W2, b2.reshape(1, 1))

    impt = imp.reshape(1, n)

    out = pl.pallas_call(
        _attn_kernel,
        grid=(nblk,),
        in_specs=[blk, full((1, n)), pl.BlockSpec((BQ, 1), lambda i: (i, 0)),
                  full((n, d)), full((n, d)),
                  ipw_at(0), ipb_at(0), full((d, d)), full((1, d))],
        out_specs=blk,
        out_shape=jax.ShapeDtypeStruct((n, d), jnp.float32),
    )(q2, impt, imp, kp, vp, in_proj_w, ipb, out_w, out_b.reshape(1, d))

    return out.reshape(batch, seq, d)


# speedup vs baseline: 2.0950x; 1.1619x over previous
import math

import jax
import jax.numpy as jnp
from jax import lax
from jax.experimental import pallas as pl
from jax.experimental.pallas import tpu as pltpu

H = 4
SPARSITY_RATIO = 0.1
MEMORY_PRESSURE = 0.5
THRESH = SPARSITY_RATIO * (1.0 + MEMORY_PRESSURE)
NEG = -1e30
BQ = 512

_DNT = (((1,), (1,)), ((), ()))


def _mmt(x, w):
    return lax.dot_general(x, w, _DNT, preferred_element_type=jnp.float32)


def _fused_kernel(k_ref, v_ref, q_ref, wk_ref, wv_ref, wq_ref,
                  w1_ref, b1_ref, w2_ref, b2_ref, ipb_ref, ow_ref, ob_ref,
                  out_ref, kp_s, vp_s, imp_s):
    i = pl.program_id(0)
    nblk = pl.num_programs(0) // 2
    d = k_ref.shape[1]
    hd = d // H

    @pl.when(i < nblk)
    def _proj_phase():
        rows = pl.ds(i * BQ, BQ)
        kp_s[rows, :] = _mmt(k_ref[...], wk_ref[...]) + ipb_ref[1]
        vp_s[rows, :] = _mmt(v_ref[...], wv_ref[...]) + ipb_ref[2]
        hid = jnp.maximum(_mmt(q_ref[...], w1_ref[...]) + b1_ref[...], 0.0)
        logit = _mmt(w2_ref[...], hid) + b2_ref[...]
        imp_s[:, pl.ds(i * BQ, BQ)] = jax.nn.sigmoid(logit)

    @pl.when(i >= nblk)
    def _attn_phase():
        j = i - nblk

        imp_row = imp_s[...]
        validk = (imp_row > THRESH).astype(jnp.float32)
        count = jnp.sum(validk)
        use_fb = count == 0.0
        fb_row = (lax.broadcasted_iota(jnp.int32, imp_row.shape, 1) < 32
                  ).astype(jnp.float32)
        validk = jnp.where(use_fb, fb_row, validk)
        kbias = (validk - 1.0) * (-NEG)

        impq = jnp.reshape(imp_s[0, pl.ds(j * BQ, BQ)], (BQ, 1))
        rows = lax.broadcasted_iota(jnp.int32, (BQ, 1), 0) + j * BQ
        validq = jnp.where(use_fb, (rows < 32).astype(jnp.float32),
                           (impq > THRESH).astype(jnp.float32))

        scale = 1.0 / math.sqrt(hd)
        qp = (_mmt(q_ref[...], wq_ref[...]) + ipb_ref[0]) * scale
        acc = jnp.zeros(out_ref.shape, jnp.float32)
        for h in range(H):
            sl = slice(h * hd, (h + 1) * hd)
            qh = qp[:, sl].astype(jnp.bfloat16)
            kh = kp_s[:, sl].astype(jnp.bfloat16)
            s = lax.dot_general(qh, kh, _DNT,
                                preferred_element_type=jnp.float32)
            s = s + kbias
            m = jnp.max(s, axis=1, keepdims=True)
            p = jnp.exp(s - m)
            l = jnp.sum(p, axis=1, keepdims=True)
            ctx = lax.dot_general(p.astype(jnp.bfloat16),
                                  vp_s[:, sl].astype(jnp.bfloat16),
                                  (((1,), (0,)), ((), ())),
                                  preferred_element_type=jnp.float32) / l
            acc = acc + _mmt(ctx, ow_ref[:, sl])
        out_ref[...] = (acc + ob_ref[...]) * validq


def kernel(q, k, v, W1, b1, W2, b2, in_proj_w, in_proj_b, out_w, out_b):
    batch, seq, d = q.shape
    n = batch * seq
    nblk = n // BQ
    dh = W1.shape[0]

    q2 = q.reshape(n, d)
    k2 = k.reshape(n, d)
    v2 = v.reshape(n, d)
    ipb = in_proj_b.reshape(3, 1, d)

    full = lambda shape: pl.BlockSpec(shape, lambda i: (0,) * len(shape))
    clamp = pl.BlockSpec((BQ, d), lambda i: (jnp.minimum(i, nblk - 1), 0))
    both = pl.BlockSpec((BQ, d),
                        lambda i: (jnp.where(i < nblk, i, i - nblk), 0))
    outsp = pl.BlockSpec((BQ, d),
                         lambda i: (jnp.maximum(i - nblk, 0), 0))
    ipw_at = lambda j: pl.BlockSpec((d, d), lambda i: (j, 0))

    out = pl.pallas_call(
        _fused_kernel,
        grid=(2 * nblk,),
        in_specs=[clamp, clamp, both,
                  ipw_at(1), ipw_at(2), ipw_at(0),
                  full((dh, d)), full((1, dh)), full((1, dh)), full((1, 1)),
                  full((3, 1, d)), full((d, d)), full((1, d))],
        out_specs=outsp,
        out_shape=jax.ShapeDtypeStruct((n, d), jnp.float32),
        scratch_shapes=[pltpu.VMEM((n, d), jnp.float32),
                        pltpu.VMEM((n, d), jnp.float32),
                        pltpu.VMEM((1, n), jnp.float32)],
    )(k2, v2, q2, in_proj_w, in_proj_w, in_proj_w,
      W1, b1.reshape(1, dh), W2, b2.reshape(1, 1), ipb,
      out_w, out_b.reshape(1, d))

    return out.reshape(batch, seq, d)
